# baseline (speedup 1.0000x reference)
import jax
import jax.numpy as jnp
from jax.experimental import pallas as pl
from jax.experimental.pallas import tpu as pltpu

F = 81920
B = 1024
M = 4
BB = 32
NP = B // BB
KB = 16384
NK = F // KB


def _nnue_kernel(wf_ref, bf_ref, ftwT_ref, ftb_ref, turn_ref, score_ref,
                 result_ref, l1wT_ref, l1b_ref, l2wT_ref, l2b_ref, out_ref,
                 wacc, bacc):
    k = pl.program_id(0)
    p = pl.program_id(1)
    rows = pl.ds(p * BB, BB)

    ftwT = ftwT_ref[...]
    wpart = jnp.dot(wf_ref[...], ftwT, preferred_element_type=jnp.float32)
    bpart = jnp.dot(bf_ref[...], ftwT, preferred_element_type=jnp.float32)

    @pl.when(k == 0)
    def _first():
        wacc[rows, :] = wpart
        bacc[rows, :] = bpart

    @pl.when(k > 0)
    def _rest():
        wacc[rows, :] += wpart
        bacc[rows, :] += bpart

    @pl.when(k == NK - 1)
    def _epilogue():
        ftb = ftb_ref[...]
        w = wacc[rows, :] + ftb
        b = bacc[rows, :] + ftb
        turn = turn_ref[...]
        acc_wb = jnp.concatenate([w, b], axis=1)
        acc_bw = jnp.concatenate([b, w], axis=1)
        accumulator = turn * acc_wb + (1.0 - turn) * acc_bw
        l1_x = jnp.clip(accumulator, 0.0, 1.0)
        l2_in = jnp.dot(l1_x, l1wT_ref[...],
                        preferred_element_type=jnp.float32) + l1b_ref[...]
        l2_x = jnp.clip(l2_in, 0.0, 1.0)
        model_result = jnp.dot(l2_x, l2wT_ref[...],
                               preferred_element_type=jnp.float32) + l2b_ref[...]
        wdl_m = jax.nn.sigmoid(model_result / 400.0)
        wdl_t = jax.nn.sigmoid(score_ref[...] / 400.0)
        loss = 0.5 * (wdl_m - wdl_t) ** 2 + 0.5 * (wdl_m - result_ref[...]) ** 2
        out_ref[...] = loss


def kernel(white_features, black_features, turn, score, result,
           ft_w, ft_b, l1_w, l1_b, l2_w, l2_b):
    return pl.pallas_call(
        _nnue_kernel,
        grid=(NK, NP),
        in_specs=[
            pl.BlockSpec((BB, KB), lambda k, p: (p, k)),
            pl.BlockSpec((BB, KB), lambda k, p: (p, k)),
            pl.BlockSpec((KB, M), lambda k, p: (k, 0)),
            pl.BlockSpec((1, M), lambda k, p: (0, 0)),
            pl.BlockSpec((BB, 1), lambda k, p: (p, 0)),
            pl.BlockSpec((BB, 1), lambda k, p: (p, 0)),
            pl.BlockSpec((BB, 1), lambda k, p: (p, 0)),
            pl.BlockSpec((2 * M, 8), lambda k, p: (0, 0)),
            pl.BlockSpec((1, 8), lambda k, p: (0, 0)),
            pl.BlockSpec((8, 1), lambda k, p: (0, 0)),
            pl.BlockSpec((1, 1), lambda k, p: (0, 0)),
        ],
        out_specs=pl.BlockSpec((BB, 1), lambda k, p: (p, 0)),
        out_shape=jax.ShapeDtypeStruct((B, 1), jnp.float32),
        scratch_shapes=[pltpu.VMEM((B, M), jnp.float32),
                        pltpu.VMEM((B, M), jnp.float32)],
        compiler_params=pltpu.CompilerParams(
            dimension_semantics=("arbitrary", "arbitrary")),
    )(white_features, black_features, ft_w.T, ft_b.reshape(1, M),
      turn, score, result, l1_w.T, l1_b.reshape(1, 8),
      l2_w.T, l2_b.reshape(1, 1))


# speedup vs baseline: 1.1260x; 1.1260x over previous
import jax
import jax.numpy as jnp
from jax.experimental import pallas as pl
from jax.experimental.pallas import tpu as pltpu

F = 81920
B = 1024
M = 4
BB = 128
NP = B // BB
KB = 4096
NK = F // KB


def _nnue_kernel(wf_ref, bf_ref, ftwT_ref, ftb_ref, turn_ref, score_ref,
                 result_ref, l1wT_ref, l1b_ref, l2wT_ref, l2b_ref, out_ref,
                 wacc, bacc):
    k = pl.program_id(0)
    p = pl.program_id(1)
    rows = pl.ds(p * BB, BB)

    ftwT = ftwT_ref[...]
    wpart = jnp.dot(wf_ref[...], ftwT, preferred_element_type=jnp.float32)
    bpart = jnp.dot(bf_ref[...], ftwT, preferred_element_type=jnp.float32)

    @pl.when(k == 0)
    def _first():
        wacc[rows, :] = wpart
        bacc[rows, :] = bpart

    @pl.when(k > 0)
    def _rest():
        wacc[rows, :] += wpart
        bacc[rows, :] += bpart

    @pl.when(k == NK - 1)
    def _epilogue():
        ftb = ftb_ref[...]
        w = wacc[rows, :] + ftb
        b = bacc[rows, :] + ftb
        turn = turn_ref[...]
        acc_wb = jnp.concatenate([w, b], axis=1)
        acc_bw = jnp.concatenate([b, w], axis=1)
        accumulator = turn * acc_wb + (1.0 - turn) * acc_bw
        l1_x = jnp.clip(accumulator, 0.0, 1.0)
        l2_in = jnp.dot(l1_x, l1wT_ref[...],
                        preferred_element_type=jnp.float32) + l1b_ref[...]
        l2_x = jnp.clip(l2_in, 0.0, 1.0)
        model_result = jnp.dot(l2_x, l2wT_ref[...],
                               preferred_element_type=jnp.float32) + l2b_ref[...]
        wdl_m = jax.nn.sigmoid(model_result / 400.0)
        wdl_t = jax.nn.sigmoid(score_ref[...] / 400.0)
        loss = 0.5 * (wdl_m - wdl_t) ** 2 + 0.5 * (wdl_m - result_ref[...]) ** 2
        out_ref[...] = loss


def kernel(white_features, black_features, turn, score, result,
           ft_w, ft_b, l1_w, l1_b, l2_w, l2_b):
    return pl.pallas_call(
        _nnue_kernel,
        grid=(NK, NP),
        in_specs=[
            pl.BlockSpec((BB, KB), lambda k, p: (p, k)),
            pl.BlockSpec((BB, KB), lambda k, p: (p, k)),
            pl.BlockSpec((KB, M), lambda k, p: (k, 0)),
            pl.BlockSpec((1, M), lambda k, p: (0, 0)),
            pl.BlockSpec((BB, 1), lambda k, p: (p, 0)),
            pl.BlockSpec((BB, 1), lambda k, p: (p, 0)),
            pl.BlockSpec((BB, 1), lambda k, p: (p, 0)),
            pl.BlockSpec((2 * M, 8), lambda k, p: (0, 0)),
            pl.BlockSpec((1, 8), lambda k, p: (0, 0)),
            pl.BlockSpec((8, 1), lambda k, p: (0, 0)),
            pl.BlockSpec((1, 1), lambda k, p: (0, 0)),
        ],
        out_specs=pl.BlockSpec((BB, 1), lambda k, p: (p, 0)),
        out_shape=jax.ShapeDtypeStruct((B, 1), jnp.float32),
        scratch_shapes=[pltpu.VMEM((B, M), jnp.float32),
                        pltpu.VMEM((B, M), jnp.float32)],
        compiler_params=pltpu.CompilerParams(
            dimension_semantics=("arbitrary", "arbitrary")),
    )(white_features, black_features, ft_w.T, ft_b.reshape(1, M),
      turn, score, result, l1_w.T, l1_b.reshape(1, 8),
      l2_w.T, l2_b.reshape(1, 1))


# speedup vs baseline: 1.4009x; 1.2441x over previous
import jax
import jax.numpy as jnp
from jax import lax
from jax.experimental import pallas as pl
from jax.experimental.pallas import tpu as pltpu

F = 81920
B = 1024
M = 4
BF = 1024
NSTEPS = F // BF
NBUF = 4


def _tc_main(wf_hbm, bf_hbm, ftwT_hbm, wout_ref, bout_ref,
             wbufs, bbufs, tbufs, wacc, bacc, sems):

    def start(step, b):
        f0 = step * BF

        def _issue():
            pltpu.make_async_copy(
                wf_hbm.at[:, pl.ds(f0, BF)], wbufs.at[b], sems.at[0, b]).start()
            pltpu.make_async_copy(
                bf_hbm.at[:, pl.ds(f0, BF)], bbufs.at[b], sems.at[1, b]).start()
            pltpu.make_async_copy(
                ftwT_hbm.at[pl.ds(f0, BF)], tbufs.at[b], sems.at[2, b]).start()

        if isinstance(step, int):
            if step < NSTEPS:
                _issue()
        else:
            pl.when(step < NSTEPS)(_issue)

    def wait(b):
        pltpu.make_async_copy(
            wf_hbm.at[:, pl.ds(0, BF)], wbufs.at[b], sems.at[0, b]).wait()
        pltpu.make_async_copy(
            bf_hbm.at[:, pl.ds(0, BF)], bbufs.at[b], sems.at[1, b]).wait()
        pltpu.make_async_copy(
            ftwT_hbm.at[pl.ds(0, BF)], tbufs.at[b], sems.at[2, b]).wait()

    wacc[...] = jnp.zeros_like(wacc)
    bacc[...] = jnp.zeros_like(bacc)

    for b in range(NBUF):
        start(b, b)

    def group(g, carry):
        step0 = g * NBUF
        for b in range(NBUF):
            wait(b)
            ftwT = tbufs[b]
            wacc[...] += jnp.dot(wbufs[b], ftwT,
                                 preferred_element_type=jnp.float32)
            bacc[...] += jnp.dot(bbufs[b], ftwT,
                                 preferred_element_type=jnp.float32)
            start(step0 + b + NBUF, b)
        return carry

    lax.fori_loop(0, NSTEPS // NBUF, group, 0)
    wout_ref[...] = wacc[...]
    bout_ref[...] = bacc[...]


def _epilogue(wtc_ref, btc_ref, ftb_ref, turn_ref, score_ref, result_ref,
              l1wT_ref, l1b_ref, l2wT_ref, l2b_ref, out_ref):
    ftb = ftb_ref[...]
    w = wtc_ref[...] + ftb
    b = btc_ref[...] + ftb
    turn = turn_ref[...]
    acc_wb = jnp.concatenate([w, b], axis=1)
    acc_bw = jnp.concatenate([b, w], axis=1)
    accumulator = turn * acc_wb + (1.0 - turn) * acc_bw
    l1_x = jnp.clip(accumulator, 0.0, 1.0)
    l2_in = jnp.dot(l1_x, l1wT_ref[...],
                    preferred_element_type=jnp.float32) + l1b_ref[...]
    l2_x = jnp.clip(l2_in, 0.0, 1.0)
    model_result = jnp.dot(l2_x, l2wT_ref[...],
                           preferred_element_type=jnp.float32) + l2b_ref[...]
    wdl_m = jax.nn.sigmoid(model_result / 400.0)
    wdl_t = jax.nn.sigmoid(score_ref[...] / 400.0)
    loss = 0.5 * (wdl_m - wdl_t) ** 2 + 0.5 * (wdl_m - result_ref[...]) ** 2
    out_ref[...] = loss


def kernel(white_features, black_features, turn, score, result,
           ft_w, ft_b, l1_w, l1_b, l2_w, l2_b):
    wtc, btc = pl.pallas_call(
        _tc_main,
        in_specs=[
            pl.BlockSpec(memory_space=pltpu.HBM),
            pl.BlockSpec(memory_space=pltpu.HBM),
            pl.BlockSpec(memory_space=pltpu.HBM),
        ],
        out_specs=[pl.BlockSpec((B, M), lambda: (0, 0)),
                   pl.BlockSpec((B, M), lambda: (0, 0))],
        out_shape=[jax.ShapeDtypeStruct((B, M), jnp.float32),
                   jax.ShapeDtypeStruct((B, M), jnp.float32)],
        scratch_shapes=[
            pltpu.VMEM((NBUF, B, BF), jnp.float32),
            pltpu.VMEM((NBUF, B, BF), jnp.float32),
            pltpu.VMEM((NBUF, BF, M), jnp.float32),
            pltpu.VMEM((B, M), jnp.float32),
            pltpu.VMEM((B, M), jnp.float32),
            pltpu.SemaphoreType.DMA((3, NBUF)),
        ],
    )(white_features, black_features, ft_w.T)

    return pl.pallas_call(
        _epilogue,
        out_shape=jax.ShapeDtypeStruct((B, 1), jnp.float32),
    )(wtc, btc, ft_b.reshape(1, M), turn, score, result,
      l1_w.T, l1_b.reshape(1, 8), l2_w.T, l2_b.reshape(1, 1))
